# Initial kernel scaffold; baseline (speedup 1.0000x reference)
import jax
import jax.numpy as jnp
from jax.experimental import pallas as pl


def kernel(input_pos, k_val, v_val, k_cache, v_cache):
    raise NotImplementedError("write your pallas kernel here")


# speedup vs baseline: 1.0115x; 1.0115x over previous
import jax
import jax.numpy as jnp
from jax.experimental import pallas as pl
from jax.experimental.pallas import tpu as pltpu

_B, _H, _L, _D, _S = 8, 16, 2048, 128, 16
_GB = 1


def _body(pos_ref, kc_ref, vc_ref, kv_ref, vv_ref, ko_ref, vo_ref):
    ko_ref[...] = kc_ref[...]
    vo_ref[...] = vc_ref[...]
    for i in range(_S):
        p = pos_ref[i]
        for j in range(_GB):
            ko_ref[j, pl.ds(p, 1), :] = kv_ref[j, pl.ds(i, 1), :]
            vo_ref[j, pl.ds(p, 1), :] = vv_ref[j, pl.ds(i, 1), :]


def kernel(input_pos, k_val, v_val, k_cache, v_cache):
    bh = _B * _H
    kc = k_cache.reshape(bh, _L, _D)
    vc = v_cache.reshape(bh, _L, _D)
    kv = k_val.reshape(bh, _S, _D)
    vv = v_val.reshape(bh, _S, _D)

    cache_spec = pl.BlockSpec((_GB, _L, _D), lambda i: (i, 0, 0))
    val_spec = pl.BlockSpec((_GB, _S, _D), lambda i: (i, 0, 0))
    ko, vo = pl.pallas_call(
        _body,
        grid=(bh // _GB,),
        in_specs=[
            pl.BlockSpec(memory_space=pltpu.SMEM),
            cache_spec,
            cache_spec,
            val_spec,
            val_spec,
        ],
        out_specs=[cache_spec, cache_spec],
        out_shape=[
            jax.ShapeDtypeStruct((bh, _L, _D), jnp.float32),
            jax.ShapeDtypeStruct((bh, _L, _D), jnp.float32),
        ],
        compiler_params=pltpu.CompilerParams(
            dimension_semantics=("arbitrary",),
        ),
    )(input_pos, kc, vc, kv, vv)
    return (ko.reshape(_B, _H, _L, _D), vo.reshape(_B, _H, _L, _D))
